# baseline (speedup 1.0000x reference)
import functools

import jax
import jax.numpy as jnp
from jax import lax
from jax.experimental import pallas as pl
from jax.experimental.pallas import tpu as pltpu
from jax.experimental.pallas import tpu_sc as plsc

_R = 2048
_P = 2_000_000
_D = 8
_L = 16
_NC = 2
_NS = 16
_NW = _NC * _NS
_TILE = 128
_NT = _P // _TILE
_TPW = _NT // _NW
_XTRA = _NT - _TPW * _NW
_CT = 8
_NCHUNK = _TPW // _CT
_CP = _CT * _TILE
_GPC = _CP // _L
_TGRP = _TILE // _L


def _tables_body(nodes_ref, wts_ref, cr_ref, ci_ref):
    n = nodes_ref[...]
    w = wts_ref[...]
    for d in range(_D):
        ang = n * ((d + 1) / _D)
        cr_ref[d] = w * jnp.cos(ang)
        ci_ref[d] = w * jnp.sin(ang)


_tables = pl.pallas_call(
    _tables_body,
    out_shape=[
        jax.ShapeDtypeStruct((_D, 16, 128), jnp.float32),
        jax.ShapeDtypeStruct((_D, 16, 128), jnp.float32),
    ],
)


def _cmul(a, b):
    (ar, ai), (br, bi) = a, b
    return (ar * br - ai * bi, ar * bi + ai * br)


def _sc_body(cr_hbm, ci_hbm, idx_hbm, out_hbm,
             cr_v, ci_v, buf0_v, buf1_v, tail_v, acc_v, sem0, sem1):
    wid = lax.axis_index("s") * _NC + lax.axis_index("c")
    pltpu.sync_copy(cr_hbm, cr_v)
    pltpu.sync_copy(ci_hbm, ci_v)

    tile0 = wid * _TPW + jnp.minimum(wid, _XTRA)
    p0 = tile0 * _TILE

    def group_body(bufref, g, acc):
        off = g * _L
        cs = []
        for d in range(_D):
            vals = bufref[d, pl.ds(off, _L)]
            if d:
                vals = vals + (d * _R)
            cs.append((plsc.load_gather(cr_v, [vals]),
                       plsc.load_gather(ci_v, [vals])))
        while len(cs) > 1:
            cs = [_cmul(cs[i], cs[i + 1]) for i in range(0, len(cs), 2)]
        return acc - cs[0][0]

    def issue(c, buf, sem):
        pltpu.async_copy(idx_hbm.at[:, pl.ds(p0 + c * _CP, _CP)], buf, sem)

    def drain(buf, sem):
        pltpu.make_async_copy(idx_hbm.at[:, pl.ds(0, _CP)], buf, sem).wait()

    def process(buf, acc):
        return plsc.parallel_loop(0, _GPC, unroll=4, carry=acc)(
            lambda g, a: group_body(buf, g, a))

    issue(0, buf0_v, sem0)
    issue(1, buf1_v, sem1)

    def pair_body(k, acc):
        drain(buf0_v, sem0)
        acc = process(buf0_v, acc)

        @pl.when(2 * k + 2 < _NCHUNK)
        def _():
            issue(2 * k + 2, buf0_v, sem0)

        drain(buf1_v, sem1)
        acc = process(buf1_v, acc)

        @pl.when(2 * k + 3 < _NCHUNK)
        def _():
            issue(2 * k + 3, buf1_v, sem1)

        return acc

    acc = lax.fori_loop(0, _NCHUNK // 2, pair_body,
                        jnp.zeros((_L,), jnp.float32))
    drain(buf0_v, sem0)
    acc = process(buf0_v, acc)

    tp = jnp.minimum(tile0 + _TPW, _NT - 1) * _TILE
    pltpu.sync_copy(idx_hbm.at[:, pl.ds(tp, _TILE)], tail_v)
    tacc = lax.fori_loop(
        0, _TGRP, lambda g, a: group_body(tail_v, g, a),
        jnp.zeros((_L,), jnp.float32))
    acc = acc + jnp.where(wid < _XTRA, tacc, jnp.zeros((_L,), jnp.float32))

    acc_v[...] = acc
    pltpu.sync_copy(acc_v, out_hbm.at[wid])


_sc_compute = functools.partial(
    pl.kernel,
    out_type=jax.ShapeDtypeStruct((_NW, _L), jnp.float32),
    mesh=plsc.VectorSubcoreMesh(core_axis_name="c", subcore_axis_name="s"),
    compiler_params=pltpu.CompilerParams(needs_layout_passes=False),
    scratch_types=[
        pltpu.VMEM((_D * _R,), jnp.float32),
        pltpu.VMEM((_D * _R,), jnp.float32),
        pltpu.VMEM((_D, _CP), jnp.int32),
        pltpu.VMEM((_D, _CP), jnp.int32),
        pltpu.VMEM((_D, _TILE), jnp.int32),
        pltpu.VMEM((_L,), jnp.float32),
        pltpu.SemaphoreType.DMA,
        pltpu.SemaphoreType.DMA,
    ],
)(_sc_body)


def kernel(rule_nodes, rule_weights, point_rule_indices):
    idx_t = point_rule_indices.astype(jnp.int32).T
    cr, ci = _tables(rule_nodes.reshape(16, 128), rule_weights.reshape(16, 128))
    parts = _sc_compute(cr.reshape(-1), ci.reshape(-1), idx_t)
    return jnp.sum(parts)


# speedup vs baseline: 1.2610x; 1.2610x over previous
import functools

import jax
import jax.numpy as jnp
from jax import lax
from jax.experimental import pallas as pl
from jax.experimental.pallas import tpu as pltpu
from jax.experimental.pallas import tpu_sc as plsc

_R = 2048
_P = 2_000_000
_D = 8
_L = 16
_NC = 2
_NS = 16
_NW = _NC * _NS
_TILE = 128
_NT = _P // _TILE
_TPW = _NT // _NW
_XTRA = _NT - _TPW * _NW
_CT = 8
_NCHUNK = _TPW // _CT
_CP = _CT * _TILE
_GPC = _CP // _L
_TGRP = _TILE // _L


def _tables_body(nodes_ref, wts_ref, tab_ref):
    n = nodes_ref[...]
    w = wts_ref[...]
    for d in range(_D):
        ang = n * ((d + 1) / _D)
        cr = (w * jnp.cos(ang)).astype(jnp.bfloat16)
        ci = (w * jnp.sin(ang)).astype(jnp.bfloat16)
        crw = lax.bitcast_convert_type(cr, jnp.uint16).astype(jnp.uint32)
        ciw = lax.bitcast_convert_type(ci, jnp.uint16).astype(jnp.uint32)
        tab_ref[d] = ((crw << 16) | ciw).astype(jnp.int32)


_tables = pl.pallas_call(
    _tables_body,
    out_shape=jax.ShapeDtypeStruct((_D, 16, 128), jnp.int32),
)


def _cmul(a, b):
    (ar, ai), (br, bi) = a, b
    return (ar * br - ai * bi, ar * bi + ai * br)


def _sc_body(tab_hbm, idx_hbm, out_hbm,
             tab_v, buf0_v, buf1_v, tail_v, acc_v, sem0, sem1):
    wid = lax.axis_index("s") * _NC + lax.axis_index("c")
    pltpu.sync_copy(tab_hbm, tab_v)

    tile0 = wid * _TPW + jnp.minimum(wid, _XTRA)
    p0 = tile0 * _TILE
    sems = (sem0, sem1)
    bufs = (buf0_v, buf1_v)
    copies = [None, None]
    copies[0] = pltpu.async_copy(idx_hbm.at[:, pl.ds(p0, _CP)], buf0_v, sem0)

    def group_body(bufref, g, acc):
        off = g * _L
        cs = []
        for d in range(_D):
            vals = bufref[d, pl.ds(off, _L)]
            if d:
                vals = vals + (d * _R)
            word = plsc.load_gather(tab_v, [vals])
            c = plsc.bitcast(word & jnp.int32(-65536), jnp.float32)
            s = plsc.bitcast(word << 16, jnp.float32)
            cs.append((c, s))
        while len(cs) > 2:
            cs = [_cmul(cs[i], cs[i + 1]) for i in range(0, len(cs), 2)]
        (ar, ai), (br, bi) = cs
        return acc - (ar * br - ai * bi)

    acc = jnp.zeros((_L,), jnp.float32)
    for ch in range(_NCHUNK):
        nxt = ch + 1
        if nxt < _NCHUNK:
            copies[nxt % 2] = pltpu.async_copy(
                idx_hbm.at[:, pl.ds(p0 + nxt * _CP, _CP)],
                bufs[nxt % 2], sems[nxt % 2])
        copies[ch % 2].wait()
        bref = bufs[ch % 2]
        acc = lax.fori_loop(0, _GPC, lambda g, a: group_body(bref, g, a), acc)

    tp = jnp.minimum(tile0 + _TPW, _NT - 1) * _TILE
    pltpu.sync_copy(idx_hbm.at[:, pl.ds(tp, _TILE)], tail_v)
    tacc = lax.fori_loop(
        0, _TGRP, lambda g, a: group_body(tail_v, g, a),
        jnp.zeros((_L,), jnp.float32))
    acc = acc + jnp.where(wid < _XTRA, tacc, jnp.zeros((_L,), jnp.float32))

    acc_v[...] = acc
    pltpu.sync_copy(acc_v, out_hbm.at[wid])


_sc_compute = functools.partial(
    pl.kernel,
    out_type=jax.ShapeDtypeStruct((_NW, _L), jnp.float32),
    mesh=plsc.VectorSubcoreMesh(core_axis_name="c", subcore_axis_name="s"),
    compiler_params=pltpu.CompilerParams(needs_layout_passes=False),
    scratch_types=[
        pltpu.VMEM((_D * _R,), jnp.int32),
        pltpu.VMEM((_D, _CP), jnp.int32),
        pltpu.VMEM((_D, _CP), jnp.int32),
        pltpu.VMEM((_D, _TILE), jnp.int32),
        pltpu.VMEM((_L,), jnp.float32),
        pltpu.SemaphoreType.DMA,
        pltpu.SemaphoreType.DMA,
    ],
)(_sc_body)


def kernel(rule_nodes, rule_weights, point_rule_indices):
    idx_t = point_rule_indices.astype(jnp.int32).T
    tab = _tables(rule_nodes.reshape(16, 128), rule_weights.reshape(16, 128))
    parts = _sc_compute(tab.reshape(-1), idx_t)
    return jnp.sum(parts)


# speedup vs baseline: 1.3578x; 1.0768x over previous
import functools

import jax
import jax.numpy as jnp
from jax import lax
from jax.experimental import pallas as pl
from jax.experimental.pallas import tpu as pltpu
from jax.experimental.pallas import tpu_sc as plsc

_R = 2048
_P = 2_000_000
_D = 8
_L = 16
_NC = 2
_NS = 16
_NW = _NC * _NS
_TILE = 128
_NT = _P // _TILE
_TPW = _NT // _NW
_XTRA = _NT - _TPW * _NW
_CT = 32
_NCHUNK = 15
_CP = _CT * _TILE
_GPC = _CP // _L
_ST = _TPW - _NCHUNK * _CT
_SP = _ST * _TILE
_SGRP = _SP // _L
_TGRP = _TILE // _L


def _tables_body(nodes_ref, wts_ref, tab_ref):
    n = nodes_ref[...]
    w = wts_ref[...]
    for d in range(_D):
        ang = n * ((d + 1) / _D)
        cr = (w * jnp.cos(ang)).astype(jnp.bfloat16)
        ci = (w * jnp.sin(ang)).astype(jnp.bfloat16)
        crw = lax.bitcast_convert_type(cr, jnp.uint16).astype(jnp.uint32)
        ciw = lax.bitcast_convert_type(ci, jnp.uint16).astype(jnp.uint32)
        tab_ref[d] = ((crw << 16) | ciw).astype(jnp.int32)


_tables = pl.pallas_call(
    _tables_body,
    out_shape=jax.ShapeDtypeStruct((_D, 16, 128), jnp.int32),
)


def _cmul(a, b):
    (ar, ai), (br, bi) = a, b
    return (ar * br - ai * bi, ar * bi + ai * br)


def _sc_body(tab_hbm, idx_hbm, out_hbm,
             t0_v, t1_v, t2_v, t3_v, t4_v, t5_v, t6_v, t7_v,
             buf0_v, buf1_v, tail_v, acc_v, sem0, sem1):
    wid = lax.axis_index("s") * _NC + lax.axis_index("c")
    tabs = (t0_v, t1_v, t2_v, t3_v, t4_v, t5_v, t6_v, t7_v)
    for d in range(_D):
        pltpu.sync_copy(tab_hbm.at[pl.ds(d * _R, _R)], tabs[d])

    tile0 = wid * _TPW + jnp.minimum(wid, _XTRA)
    p0 = tile0 * _TILE
    sems = (sem0, sem1)
    bufs = (buf0_v, buf1_v)
    copies = [None, None]
    copies[0] = pltpu.async_copy(idx_hbm.at[:, pl.ds(p0, _CP)], buf0_v, sem0)

    def group_body(bufref, g, acc):
        off = g * _L
        cs = []
        for d in range(_D):
            vals = bufref[d, pl.ds(off, _L)]
            word = plsc.load_gather(tabs[d], [vals])
            c = plsc.bitcast(word & jnp.int32(-65536), jnp.float32)
            s = plsc.bitcast(word << 16, jnp.float32)
            cs.append((c, s))
        while len(cs) > 2:
            cs = [_cmul(cs[i], cs[i + 1]) for i in range(0, len(cs), 2)]
        (ar, ai), (br, bi) = cs
        return acc - (ar * br - ai * bi)

    def process(bufref, ngroups, acc):
        return plsc.parallel_loop(0, ngroups, unroll=2, carry=acc)(
            lambda g, a: group_body(bufref, g, a))

    acc = jnp.zeros((_L,), jnp.float32)
    for ch in range(_NCHUNK):
        if ch + 1 < _NCHUNK:
            copies[(ch + 1) % 2] = pltpu.async_copy(
                idx_hbm.at[:, pl.ds(p0 + (ch + 1) * _CP, _CP)],
                bufs[(ch + 1) % 2], sems[(ch + 1) % 2])
        elif ch + 1 == _NCHUNK:
            copies[(ch + 1) % 2] = pltpu.async_copy(
                idx_hbm.at[:, pl.ds(p0 + _NCHUNK * _CP, _SP)],
                bufs[(ch + 1) % 2].at[:, pl.ds(0, _SP)], sems[(ch + 1) % 2])
        copies[ch % 2].wait()
        acc = process(bufs[ch % 2], _GPC, acc)
    copies[_NCHUNK % 2].wait()
    acc = process(bufs[_NCHUNK % 2], _SGRP, acc)

    tp = jnp.minimum(tile0 + _TPW, _NT - 1) * _TILE
    pltpu.sync_copy(idx_hbm.at[:, pl.ds(tp, _TILE)], tail_v)
    tacc = process(tail_v, _TGRP, jnp.zeros((_L,), jnp.float32))
    acc = acc + jnp.where(wid < _XTRA, tacc, jnp.zeros((_L,), jnp.float32))

    acc_v[...] = acc
    pltpu.sync_copy(acc_v, out_hbm.at[wid])


_sc_compute = functools.partial(
    pl.kernel,
    out_type=jax.ShapeDtypeStruct((_NW, _L), jnp.float32),
    mesh=plsc.VectorSubcoreMesh(core_axis_name="c", subcore_axis_name="s"),
    compiler_params=pltpu.CompilerParams(needs_layout_passes=False),
    scratch_types=(
        [pltpu.VMEM((_R,), jnp.int32) for _ in range(_D)]
        + [
            pltpu.VMEM((_D, _CP), jnp.int32),
            pltpu.VMEM((_D, _CP), jnp.int32),
            pltpu.VMEM((_D, _TILE), jnp.int32),
            pltpu.VMEM((_L,), jnp.float32),
            pltpu.SemaphoreType.DMA,
            pltpu.SemaphoreType.DMA,
        ]
    ),
)(_sc_body)


def kernel(rule_nodes, rule_weights, point_rule_indices):
    idx_t = point_rule_indices.astype(jnp.int32).T
    tab = _tables(rule_nodes.reshape(16, 128), rule_weights.reshape(16, 128))
    parts = _sc_compute(tab.reshape(-1), idx_t)
    return jnp.sum(parts)
